# Initial kernel scaffold; baseline (speedup 1.0000x reference)
import jax
import jax.numpy as jnp
from jax.experimental import pallas as pl


def kernel(img, label, W_enc, b_enc, codebook, W_dec, b_dec):
    raise NotImplementedError("write your pallas kernel here")


# speedup vs baseline: 4.2391x; 4.2391x over previous
import functools

import jax
import jax.numpy as jnp
from jax import lax
from jax.experimental import pallas as pl
from jax.experimental.pallas import tpu as pltpu
from jax.experimental.pallas import tpu_sc as plsc

E = 8
D = 1024
K = 512
N = 2048
BETA = 0.25
BLK = 128
NBLK = N // BLK + E
NP = NBLK * BLK
TBLK = 256
NTB = N // TBLK


def _sc_workers():
    try:
        info = plsc.get_sparse_core_info()
        return info.num_cores, info.num_subcores
    except Exception:
        return 2, 16


def _route_body(lab_ref, pos_ref, meta_ref):
    lab = lab_ref[...]
    r128 = lax.broadcasted_iota(jnp.int32, (128, 128), 0)
    c128 = lax.broadcasted_iota(jnp.int32, (128, 128), 1)
    tri128 = (r128 < c128).astype(jnp.float32)
    r16 = lax.broadcasted_iota(jnp.int32, (16, 16), 0)
    c16 = lax.broadcasted_iota(jnp.int32, (16, 16), 1)
    tri16 = (c16 < r16).astype(jnp.float32)

    masks, ranks, cnts = [], [], []
    for e in range(E):
        m = (lab == e).astype(jnp.float32)
        pref = lax.dot_general(m, tri128, (((1,), (0,)), ((), ())),
                               preferred_element_type=jnp.float32)
        s = jnp.sum(m, axis=1, keepdims=True)
        rowpref = lax.dot_general(tri16, s, (((1,), (0,)), ((), ())),
                                  preferred_element_type=jnp.float32)
        masks.append(m)
        ranks.append(pref + rowpref)
        cnts.append(jnp.sum(m).astype(jnp.int32))

    poffs, cumblocks = [], []
    acc = jnp.int32(0)
    for e in range(E):
        poffs.append(acc * BLK)
        acc = acc + (cnts[e] + (BLK - 1)) // BLK
        cumblocks.append(acc)

    pos = jnp.zeros((16, 128), jnp.float32)
    for e in range(E):
        pos = pos + masks[e] * (ranks[e] + poffs[e].astype(jnp.float32))
    pos_ref[...] = pos.astype(jnp.int32)

    li = lax.broadcasted_iota(jnp.int32, (E, 128), 1)
    bm = jnp.zeros((E, 128), jnp.int32)
    for e in range(E):
        bm = bm + (li >= cumblocks[e]).astype(jnp.int32)
    meta = jnp.where(li < NBLK, jnp.minimum(bm, E - 1), 0)
    for e in range(E):
        meta = meta + jnp.where(li == NBLK + e, cnts[e], 0)
        meta = meta + jnp.where(li == NBLK + E + e, poffs[e], 0)
    meta_ref[...] = meta


def _route(lab2d):
    return pl.pallas_call(
        _route_body,
        out_shape=[
            jax.ShapeDtypeStruct((16, 128), jnp.int32),
            jax.ShapeDtypeStruct((E, 128), jnp.int32),
        ],
    )(lab2d)


def _dispatch(img, pos):
    nc, ns = _sc_workers()
    nw = nc * ns
    tpw = N // nw
    mesh = plsc.VectorSubcoreMesh(core_axis_name="c", subcore_axis_name="s")

    @functools.partial(
        pl.kernel, mesh=mesh,
        out_type=jax.ShapeDtypeStruct((NP, D), jnp.float32),
        scratch_types=[
            pltpu.VMEM((tpw,), jnp.int32),
            pltpu.VMEM((tpw, D), jnp.float32),
            pltpu.SemaphoreType.DMA,
        ],
    )
    def body(img_hbm, pos_hbm, out_hbm, idx_v, rows_v, sem):
        wid = lax.axis_index("s") * nc + lax.axis_index("c")
        base = wid * tpw
        pltpu.sync_copy(pos_hbm.at[pl.ds(base, tpw)], idx_v)
        pltpu.sync_copy(img_hbm.at[pl.ds(base, tpw)], rows_v)
        pltpu.async_copy(rows_v, out_hbm.at[idx_v], sem).wait()

    return body(img, pos)


def _gather_rows(table, idx):
    nc, ns = _sc_workers()
    nw = nc * ns
    tpw = N // nw
    mesh = plsc.VectorSubcoreMesh(core_axis_name="c", subcore_axis_name="s")

    @functools.partial(
        pl.kernel, mesh=mesh,
        out_type=jax.ShapeDtypeStruct((N, D), jnp.float32),
        scratch_types=[
            pltpu.VMEM((tpw,), jnp.int32),
            pltpu.VMEM((tpw, D), jnp.float32),
            pltpu.SemaphoreType.DMA,
        ],
    )
    def body(tab_hbm, idx_hbm, out_hbm, idx_v, rows_v, sem):
        wid = lax.axis_index("s") * nc + lax.axis_index("c")
        base = wid * tpw
        pltpu.sync_copy(idx_hbm.at[pl.ds(base, tpw)], idx_v)
        pltpu.async_copy(tab_hbm.at[idx_v], rows_v, sem).wait()
        pltpu.sync_copy(rows_v, out_hbm.at[pl.ds(base, tpw)])

    return body(table, idx)


def _encode_body(bmap_ref, x_ref, w_ref, b_ref, o_ref):
    e = bmap_ref[pl.program_id(0)]
    acc = lax.dot_general(x_ref[...], w_ref[0], (((1,), (0,)), ((), ())),
                          preferred_element_type=jnp.float32)
    o_ref[...] = acc + b_ref[pl.ds(e, 1), :]


def _encode(bmap, simg, W_enc, b_enc):
    grid_spec = pltpu.PrefetchScalarGridSpec(
        num_scalar_prefetch=1,
        grid=(NBLK,),
        in_specs=[
            pl.BlockSpec((BLK, D), lambda i, bmap: (i, 0)),
            pl.BlockSpec((1, D, D), lambda i, bmap: (bmap[i], 0, 0)),
            pl.BlockSpec((E, D), lambda i, bmap: (0, 0)),
        ],
        out_specs=pl.BlockSpec((BLK, D), lambda i, bmap: (i, 0)),
    )
    return pl.pallas_call(
        _encode_body,
        grid_spec=grid_spec,
        out_shape=jax.ShapeDtypeStruct((NP, D), jnp.float32),
    )(bmap, simg, W_enc, b_enc)


def _vq_body(x_ref, cb_ref, idx_ref):
    x = x_ref[...]
    cb = cb_ref[...]
    xsq = jnp.sum(x * x, axis=1, keepdims=True)
    prod = lax.dot_general(x, cb, (((1,), (1,)), ((), ())),
                           preferred_element_type=jnp.float32)
    cbsq = jnp.sum(cb * cb, axis=1)[None, :]
    d2 = xsq - 2.0 * prod + cbsq
    mv = jnp.min(d2, axis=1, keepdims=True)
    li = lax.broadcasted_iota(jnp.int32, (TBLK, K), 1)
    idx = jnp.min(jnp.where(d2 <= mv, li, K), axis=1, keepdims=True)
    idx_ref[...] = jnp.minimum(idx, K - 1)


def _vq(enc, codebook):
    return pl.pallas_call(
        _vq_body,
        grid=(NTB,),
        in_specs=[
            pl.BlockSpec((TBLK, D), lambda i: (i, 0)),
            pl.BlockSpec((K, D), lambda i: (0, 0)),
        ],
        out_specs=pl.BlockSpec((TBLK, 1), lambda i: (i, 0)),
        out_shape=jax.ShapeDtypeStruct((N, 1), jnp.int32),
    )(enc, codebook)


def _decode_body(cnt_ref, q_ref, img_ref, enc_ref, lab_ref, wd_ref, bd_ref,
                 out_ref, loss_ref, sse_acc, s_acc):
    i = pl.program_id(0)

    @pl.when(i == 0)
    def _():
        sse_acc[0] = jnp.float32(0.0)
        for e in range(E):
            s_acc[e] = jnp.float32(0.0)

    q = q_ref[...]
    dec = lax.dot_general(q, wd_ref[...], (((1,), (0,)), ((), ())),
                          preferred_element_type=jnp.float32) + bd_ref[...]
    out_ref[...] = dec

    diff = dec - img_ref[...]
    sse_acc[0] += jnp.sum(diff * diff)

    qd = enc_ref[...] - q
    qtok = jnp.sum(qd * qd, axis=1, keepdims=True)
    labv = lab_ref[...]
    for e in range(E):
        m = (labv == e).astype(jnp.float32)
        s_acc[e] += jnp.sum(m * qtok)

    @pl.when(i == NTB - 1)
    def _():
        qloss = jnp.float32(0.0)
        for e in range(E):
            cnt = jnp.maximum(cnt_ref[e].astype(jnp.float32) * D, 1.0)
            qloss += (1.0 + BETA) * s_acc[e] / cnt
        loss_ref[0] = sse_acc[0] / (N * D) + qloss / E


def _decode_loss(counts, quant, img, enc, lab_col, W_dec, b_dec2d):
    grid_spec = pltpu.PrefetchScalarGridSpec(
        num_scalar_prefetch=1,
        grid=(NTB,),
        in_specs=[
            pl.BlockSpec((TBLK, D), lambda i, c: (i, 0)),
            pl.BlockSpec((TBLK, D), lambda i, c: (i, 0)),
            pl.BlockSpec((TBLK, D), lambda i, c: (i, 0)),
            pl.BlockSpec((TBLK, 1), lambda i, c: (i, 0)),
            pl.BlockSpec((D, D), lambda i, c: (0, 0)),
            pl.BlockSpec((1, D), lambda i, c: (0, 0)),
        ],
        out_specs=[
            pl.BlockSpec((TBLK, D), lambda i, c: (i, 0)),
            pl.BlockSpec(memory_space=pltpu.SMEM),
        ],
        scratch_shapes=[
            pltpu.SMEM((1,), jnp.float32),
            pltpu.SMEM((E,), jnp.float32),
        ],
    )
    return pl.pallas_call(
        _decode_body,
        grid_spec=grid_spec,
        out_shape=[
            jax.ShapeDtypeStruct((N, D), jnp.float32),
            jax.ShapeDtypeStruct((1,), jnp.float32),
        ],
    )(counts, quant, img, enc, lab_col, W_dec, b_dec2d)


def kernel(img, label, W_enc, b_enc, codebook, W_dec, b_dec):
    lab = label.astype(jnp.int32)
    pos2d, meta = _route(lab.reshape(16, 128))
    pos = pos2d.reshape(N)
    bmap = meta[0, :NBLK]
    counts = meta[0, NBLK:NBLK + E]

    simg = _dispatch(img, pos)
    enc_sorted = _encode(bmap, simg, W_enc, b_enc)
    enc = _gather_rows(enc_sorted, pos)
    idx = _vq(enc, codebook).reshape(N)
    quant = _gather_rows(codebook, idx)
    out, loss = _decode_loss(counts, quant, img, enc, lab.reshape(N, 1),
                             W_dec, b_dec.reshape(1, D))
    return out, loss[0]


# speedup vs baseline: 4.9910x; 1.1774x over previous
import functools

import jax
import jax.numpy as jnp
from jax import lax
from jax.experimental import pallas as pl
from jax.experimental.pallas import tpu as pltpu
from jax.experimental.pallas import tpu_sc as plsc

E = 8
D = 1024
K = 512
N = 2048
BETA = 0.25
BLK = 128
NBLK = N // BLK + E
NP = NBLK * BLK
TBLK = 256
NTB = N // TBLK


def _sc_workers():
    try:
        info = plsc.get_sparse_core_info()
        return info.num_cores, info.num_subcores
    except Exception:
        return 2, 16


def _route_body(lab_ref, pos_ref, meta_ref):
    lab = lab_ref[...]
    r128 = lax.broadcasted_iota(jnp.int32, (128, 128), 0)
    c128 = lax.broadcasted_iota(jnp.int32, (128, 128), 1)
    tri128 = (r128 < c128).astype(jnp.float32)
    r16 = lax.broadcasted_iota(jnp.int32, (16, 16), 0)
    c16 = lax.broadcasted_iota(jnp.int32, (16, 16), 1)
    tri16 = (c16 < r16).astype(jnp.float32)

    masks, ranks, cnts = [], [], []
    for e in range(E):
        m = (lab == e).astype(jnp.float32)
        pref = lax.dot_general(m, tri128, (((1,), (0,)), ((), ())),
                               preferred_element_type=jnp.float32)
        s = jnp.sum(m, axis=1, keepdims=True)
        rowpref = lax.dot_general(tri16, s, (((1,), (0,)), ((), ())),
                                  preferred_element_type=jnp.float32)
        masks.append(m)
        ranks.append(pref + rowpref)
        cnts.append(jnp.sum(m).astype(jnp.int32))

    poffs, cumblocks = [], []
    acc = jnp.int32(0)
    for e in range(E):
        poffs.append(acc * BLK)
        acc = acc + (cnts[e] + (BLK - 1)) // BLK
        cumblocks.append(acc)

    pos = jnp.zeros((16, 128), jnp.float32)
    for e in range(E):
        pos = pos + masks[e] * (ranks[e] + poffs[e].astype(jnp.float32))
    pos_ref[...] = pos.astype(jnp.int32)

    li = lax.broadcasted_iota(jnp.int32, (E, 128), 1)
    bm = jnp.zeros((E, 128), jnp.int32)
    for e in range(E):
        bm = bm + (li >= cumblocks[e]).astype(jnp.int32)
    meta = jnp.where(li < NBLK, jnp.minimum(bm, E - 1), 0)
    for e in range(E):
        meta = meta + jnp.where(li == NBLK + e, cnts[e], 0)
        meta = meta + jnp.where(li == NBLK + E + e, poffs[e], 0)
    meta = meta + jnp.where(li == NBLK + 2 * E, cumblocks[E - 1], 0)
    meta_ref[...] = meta


def _route(lab2d):
    return pl.pallas_call(
        _route_body,
        out_shape=[
            jax.ShapeDtypeStruct((16, 128), jnp.int32),
            jax.ShapeDtypeStruct((E, 128), jnp.int32),
        ],
    )(lab2d)


def _dispatch(img, pos):
    nc, ns = _sc_workers()
    nw = nc * ns
    tpw = N // nw
    mesh = plsc.VectorSubcoreMesh(core_axis_name="c", subcore_axis_name="s")

    @functools.partial(
        pl.kernel, mesh=mesh,
        out_type=jax.ShapeDtypeStruct((NP, D), jnp.float32),
        scratch_types=[
            pltpu.VMEM((tpw,), jnp.int32),
            pltpu.VMEM((tpw, D), jnp.float32),
            pltpu.SemaphoreType.DMA,
        ],
    )
    def body(img_hbm, pos_hbm, out_hbm, idx_v, rows_v, sem):
        wid = lax.axis_index("s") * nc + lax.axis_index("c")
        base = wid * tpw
        pltpu.sync_copy(pos_hbm.at[pl.ds(base, tpw)], idx_v)
        pltpu.sync_copy(img_hbm.at[pl.ds(base, tpw)], rows_v)
        pltpu.async_copy(rows_v, out_hbm.at[idx_v], sem).wait()

    return body(img, pos)


def _gather_rows(table, idx):
    nc, ns = _sc_workers()
    nw = nc * ns
    tpw = N // nw
    mesh = plsc.VectorSubcoreMesh(core_axis_name="c", subcore_axis_name="s")

    @functools.partial(
        pl.kernel, mesh=mesh,
        out_type=jax.ShapeDtypeStruct((N, D), jnp.float32),
        scratch_types=[
            pltpu.VMEM((tpw,), jnp.int32),
            pltpu.VMEM((tpw, D), jnp.float32),
            pltpu.SemaphoreType.DMA,
        ],
    )
    def body(tab_hbm, idx_hbm, out_hbm, idx_v, rows_v, sem):
        wid = lax.axis_index("s") * nc + lax.axis_index("c")
        base = wid * tpw
        pltpu.sync_copy(idx_hbm.at[pl.ds(base, tpw)], idx_v)
        pltpu.async_copy(tab_hbm.at[idx_v], rows_v, sem).wait()
        pltpu.sync_copy(rows_v, out_hbm.at[pl.ds(base, tpw)])

    return body(table, idx)


def _encode_body(meta_ref, x_ref, w_ref, b_ref, o_ref):
    i = pl.program_id(0)
    e = meta_ref[i]

    @pl.when(i < meta_ref[NBLK + 2 * E])
    def _():
        acc = lax.dot_general(x_ref[...], w_ref[0], (((1,), (0,)), ((), ())),
                              preferred_element_type=jnp.float32)
        o_ref[...] = acc + b_ref[pl.ds(e, 1), :]


def _encode(bmap, simg, W_enc, b_enc):
    grid_spec = pltpu.PrefetchScalarGridSpec(
        num_scalar_prefetch=1,
        grid=(NBLK,),
        in_specs=[
            pl.BlockSpec((BLK, D), lambda i, bmap: (i, 0)),
            pl.BlockSpec((1, D, D), lambda i, bmap: (bmap[i], 0, 0)),
            pl.BlockSpec((E, D), lambda i, bmap: (0, 0)),
        ],
        out_specs=pl.BlockSpec((BLK, D), lambda i, bmap: (i, 0)),
    )
    return pl.pallas_call(
        _encode_body,
        grid_spec=grid_spec,
        out_shape=jax.ShapeDtypeStruct((NP, D), jnp.float32),
    )(bmap, simg, W_enc, b_enc)


def _vqdec_body(cnt_ref, x_ref, cb_ref, img_ref, lab_ref, wd_ref, bd_ref,
                out_ref, loss_ref, sse_acc, s_acc):
    i = pl.program_id(0)

    @pl.when(i == 0)
    def _():
        sse_acc[0] = jnp.float32(0.0)
        for e in range(E):
            s_acc[e] = jnp.float32(0.0)

    x = x_ref[...]
    cb = cb_ref[...]
    xsq = jnp.sum(x * x, axis=1, keepdims=True)
    prod = lax.dot_general(x, cb, (((1,), (1,)), ((), ())),
                           preferred_element_type=jnp.float32)
    cbsq = jnp.sum(cb * cb, axis=1)[None, :]
    d2 = xsq - 2.0 * prod + cbsq
    mv = jnp.min(d2, axis=1, keepdims=True)
    li = lax.broadcasted_iota(jnp.int32, (TBLK, K), 1)
    idx = jnp.min(jnp.where(d2 <= mv, li, K), axis=1, keepdims=True)
    oh = (li == idx).astype(jnp.float32)
    quant = lax.dot_general(oh, cb, (((1,), (0,)), ((), ())),
                            preferred_element_type=jnp.float32)

    dec = lax.dot_general(quant, wd_ref[...], (((1,), (0,)), ((), ())),
                          preferred_element_type=jnp.float32) + bd_ref[...]
    out_ref[...] = dec

    diff = dec - img_ref[...]
    sse_acc[0] += jnp.sum(diff * diff)

    qd = x - quant
    qtok = jnp.sum(qd * qd, axis=1, keepdims=True)
    labv = lab_ref[...]
    for e in range(E):
        m = (labv == e).astype(jnp.float32)
        s_acc[e] += jnp.sum(m * qtok)

    @pl.when(i == NTB - 1)
    def _():
        qloss = jnp.float32(0.0)
        for e in range(E):
            cnt = jnp.maximum(cnt_ref[e].astype(jnp.float32) * D, 1.0)
            qloss += (1.0 + BETA) * s_acc[e] / cnt
        loss_ref[0] = sse_acc[0] / (N * D) + qloss / E


def _vq_decode_loss(counts, enc, codebook, img, lab_col, W_dec, b_dec2d):
    grid_spec = pltpu.PrefetchScalarGridSpec(
        num_scalar_prefetch=1,
        grid=(NTB,),
        in_specs=[
            pl.BlockSpec((TBLK, D), lambda i, c: (i, 0)),
            pl.BlockSpec((K, D), lambda i, c: (0, 0)),
            pl.BlockSpec((TBLK, D), lambda i, c: (i, 0)),
            pl.BlockSpec((TBLK, 1), lambda i, c: (i, 0)),
            pl.BlockSpec((D, D), lambda i, c: (0, 0)),
            pl.BlockSpec((1, D), lambda i, c: (0, 0)),
        ],
        out_specs=[
            pl.BlockSpec((TBLK, D), lambda i, c: (i, 0)),
            pl.BlockSpec(memory_space=pltpu.SMEM),
        ],
        scratch_shapes=[
            pltpu.SMEM((1,), jnp.float32),
            pltpu.SMEM((E,), jnp.float32),
        ],
    )
    return pl.pallas_call(
        _vqdec_body,
        grid_spec=grid_spec,
        out_shape=[
            jax.ShapeDtypeStruct((N, D), jnp.float32),
            jax.ShapeDtypeStruct((1,), jnp.float32),
        ],
    )(counts, enc, codebook, img, lab_col, W_dec, b_dec2d)


def kernel(img, label, W_enc, b_enc, codebook, W_dec, b_dec):
    lab = label.astype(jnp.int32)
    pos2d, meta = _route(lab.reshape(16, 128))
    pos = pos2d.reshape(N)
    meta0 = meta[0]
    counts = meta0[NBLK:NBLK + E]

    simg = _dispatch(img, pos)
    enc_sorted = _encode(meta0, simg, W_enc, b_enc)
    enc = _gather_rows(enc_sorted, pos)
    out, loss = _vq_decode_loss(counts, enc, codebook, img,
                                lab.reshape(N, 1), W_dec, b_dec.reshape(1, D))
    return out, loss[0]


# speedup vs baseline: 5.2963x; 1.0612x over previous
import functools

import jax
import jax.numpy as jnp
from jax import lax
from jax.experimental import pallas as pl
from jax.experimental.pallas import tpu as pltpu
from jax.experimental.pallas import tpu_sc as plsc

E = 8
D = 1024
K = 512
N = 2048
BETA = 0.25
BLK = 128
NBLK = N // BLK + E
NP = NBLK * BLK


def _sc_workers():
    try:
        info = plsc.get_sparse_core_info()
        return info.num_cores, info.num_subcores
    except Exception:
        return 2, 16


def _route_body(lab_ref, pos_ref, meta_ref):
    lab = lab_ref[...]
    r128 = lax.broadcasted_iota(jnp.int32, (128, 128), 0)
    c128 = lax.broadcasted_iota(jnp.int32, (128, 128), 1)
    tri128 = (r128 < c128).astype(jnp.float32)
    r16 = lax.broadcasted_iota(jnp.int32, (16, 16), 0)
    c16 = lax.broadcasted_iota(jnp.int32, (16, 16), 1)
    tri16 = (c16 < r16).astype(jnp.float32)

    masks, ranks, cnts = [], [], []
    for e in range(E):
        m = (lab == e).astype(jnp.float32)
        pref = lax.dot_general(m, tri128, (((1,), (0,)), ((), ())),
                               preferred_element_type=jnp.float32)
        s = jnp.sum(m, axis=1, keepdims=True)
        rowpref = lax.dot_general(tri16, s, (((1,), (0,)), ((), ())),
                                  preferred_element_type=jnp.float32)
        masks.append(m)
        ranks.append(pref + rowpref)
        cnts.append(jnp.sum(m).astype(jnp.int32))

    poffs, cumblocks = [], []
    acc = jnp.int32(0)
    for e in range(E):
        poffs.append(acc * BLK)
        acc = acc + (cnts[e] + (BLK - 1)) // BLK
        cumblocks.append(acc)

    pos = jnp.zeros((16, 128), jnp.float32)
    for e in range(E):
        pos = pos + masks[e] * (ranks[e] + poffs[e].astype(jnp.float32))
    pos_ref[...] = pos.astype(jnp.int32)

    li = lax.broadcasted_iota(jnp.int32, (E, 128), 1)
    bm = jnp.zeros((E, 128), jnp.int32)
    for e in range(E):
        bm = bm + (li >= cumblocks[e]).astype(jnp.int32)
    meta = jnp.where(li < NBLK, jnp.minimum(bm, E - 1), 0)
    for e in range(E):
        meta = meta + jnp.where(li == NBLK + e, cnts[e], 0)
        meta = meta + jnp.where(li == NBLK + E + e, poffs[e], 0)
    meta = meta + jnp.where(li == NBLK + 2 * E, cumblocks[E - 1], 0)
    meta_ref[...] = meta


def _route(lab2d):
    return pl.pallas_call(
        _route_body,
        out_shape=[
            jax.ShapeDtypeStruct((16, 128), jnp.int32),
            jax.ShapeDtypeStruct((E, 128), jnp.int32),
        ],
    )(lab2d)


def _dispatch(img, pos):
    nc, ns = _sc_workers()
    nw = nc * ns
    tpw = N // nw
    mesh = plsc.VectorSubcoreMesh(core_axis_name="c", subcore_axis_name="s")

    @functools.partial(
        pl.kernel, mesh=mesh,
        out_type=jax.ShapeDtypeStruct((NP, D), jnp.float32),
        scratch_types=[
            pltpu.VMEM((tpw,), jnp.int32),
            pltpu.VMEM((tpw, D), jnp.float32),
            pltpu.SemaphoreType.DMA,
        ],
    )
    def body(img_hbm, pos_hbm, out_hbm, idx_v, rows_v, sem):
        wid = lax.axis_index("s") * nc + lax.axis_index("c")
        base = wid * tpw
        pltpu.sync_copy(pos_hbm.at[pl.ds(base, tpw)], idx_v)
        pltpu.sync_copy(img_hbm.at[pl.ds(base, tpw)], rows_v)
        pltpu.async_copy(rows_v, out_hbm.at[idx_v], sem).wait()

    return body(img, pos)


def _gather_rows(table, idx):
    nc, ns = _sc_workers()
    nw = nc * ns
    tpw = N // nw
    mesh = plsc.VectorSubcoreMesh(core_axis_name="c", subcore_axis_name="s")

    @functools.partial(
        pl.kernel, mesh=mesh,
        out_type=jax.ShapeDtypeStruct((N, D), jnp.float32),
        scratch_types=[
            pltpu.VMEM((tpw,), jnp.int32),
            pltpu.VMEM((tpw, D), jnp.float32),
            pltpu.SemaphoreType.DMA,
        ],
    )
    def body(tab_hbm, idx_hbm, out_hbm, idx_v, rows_v, sem):
        wid = lax.axis_index("s") * nc + lax.axis_index("c")
        base = wid * tpw
        pltpu.sync_copy(idx_hbm.at[pl.ds(base, tpw)], idx_v)
        pltpu.async_copy(tab_hbm.at[idx_v], rows_v, sem).wait()
        pltpu.sync_copy(rows_v, out_hbm.at[pl.ds(base, tpw)])

    return body(table, idx)


def _main_body(m_ref, x_ref, w_ref, b_ref, cb_ref, wd_ref, bd_ref,
               dec_ref, loss_ref, sse_acc, s_acc, cbsq_ref):
    i = pl.program_id(0)
    total = m_ref[NBLK + 2 * E]

    @pl.when(i == 0)
    def _():
        sse_acc[0] = jnp.float32(0.0)
        for e in range(E):
            s_acc[e] = jnp.float32(0.0)
        cb0 = cb_ref[...]
        cbsq_ref[...] = jnp.sum(cb0 * cb0, axis=1)[None, :]

    @pl.when(i < total)
    def _():
        e = m_ref[i]
        x = x_ref[...]
        enc = lax.dot_general(x, w_ref[0], (((1,), (0,)), ((), ())),
                              preferred_element_type=jnp.float32)
        enc = enc + b_ref[pl.ds(e, 1), :]
        cb = cb_ref[...]
        xsq = jnp.sum(enc * enc, axis=1, keepdims=True)
        prod = lax.dot_general(enc, cb, (((1,), (1,)), ((), ())),
                               preferred_element_type=jnp.float32)
        d2 = xsq - 2.0 * prod + cbsq_ref[...]
        mv = jnp.min(d2, axis=1, keepdims=True)
        li = lax.broadcasted_iota(jnp.int32, (BLK, K), 1)
        idx = jnp.min(jnp.where(d2 <= mv, li, K), axis=1, keepdims=True)
        oh = (li == idx).astype(jnp.float32)
        quant = lax.dot_general(oh, cb, (((1,), (0,)), ((), ())),
                                preferred_element_type=jnp.float32)
        dec = lax.dot_general(quant, wd_ref[...], (((1,), (0,)), ((), ())),
                              preferred_element_type=jnp.float32) + bd_ref[...]
        dec_ref[...] = dec

        row = lax.broadcasted_iota(jnp.int32, (BLK, 1), 0)
        srow = i * BLK + row
        valid = (srow - m_ref[NBLK + E + e]) < m_ref[NBLK + e]
        diff = dec - x
        sse_row = jnp.sum(diff * diff, axis=1, keepdims=True)
        sse_acc[0] += jnp.sum(jnp.where(valid, sse_row, 0.0))
        qd = enc - quant
        qtok = jnp.sum(qd * qd, axis=1, keepdims=True)
        s_acc[e] += jnp.sum(jnp.where(valid, qtok, 0.0))

    @pl.when(i == NBLK - 1)
    def _():
        qloss = jnp.float32(0.0)
        for e in range(E):
            cnt = jnp.maximum(m_ref[NBLK + e].astype(jnp.float32) * D, 1.0)
            qloss += (1.0 + BETA) * s_acc[e] / cnt
        loss_ref[0] = sse_acc[0] / (N * D) + qloss / E


def _main(meta0, simg, W_enc, b_enc, codebook, W_dec, b_dec2d):
    grid_spec = pltpu.PrefetchScalarGridSpec(
        num_scalar_prefetch=1,
        grid=(NBLK,),
        in_specs=[
            pl.BlockSpec((BLK, D), lambda i, m: (i, 0)),
            pl.BlockSpec((1, D, D), lambda i, m: (m[i], 0, 0)),
            pl.BlockSpec((E, D), lambda i, m: (0, 0)),
            pl.BlockSpec((K, D), lambda i, m: (0, 0)),
            pl.BlockSpec((D, D), lambda i, m: (0, 0)),
            pl.BlockSpec((1, D), lambda i, m: (0, 0)),
        ],
        out_specs=[
            pl.BlockSpec((BLK, D), lambda i, m: (i, 0)),
            pl.BlockSpec(memory_space=pltpu.SMEM),
        ],
        scratch_shapes=[
            pltpu.SMEM((1,), jnp.float32),
            pltpu.SMEM((E,), jnp.float32),
            pltpu.VMEM((1, K), jnp.float32),
        ],
    )
    return pl.pallas_call(
        _main_body,
        grid_spec=grid_spec,
        out_shape=[
            jax.ShapeDtypeStruct((NP, D), jnp.float32),
            jax.ShapeDtypeStruct((1,), jnp.float32),
        ],
    )(meta0, simg, W_enc, b_enc, codebook, W_dec, b_dec2d)


def kernel(img, label, W_enc, b_enc, codebook, W_dec, b_dec):
    lab = label.astype(jnp.int32)
    pos2d, meta = _route(lab.reshape(16, 128))
    pos = pos2d.reshape(N)
    meta0 = meta[0]

    simg = _dispatch(img, pos)
    dec_sorted, loss = _main(meta0, simg, W_enc, b_enc, codebook,
                             W_dec, b_dec.reshape(1, D))
    out = _gather_rows(dec_sorted, pos)
    return out, loss[0]


# speedup vs baseline: 5.8973x; 1.1135x over previous
import functools

import jax
import jax.numpy as jnp
from jax import lax
from jax.experimental import pallas as pl
from jax.experimental.pallas import tpu as pltpu
from jax.experimental.pallas import tpu_sc as plsc

E = 8
D = 1024
K = 512
N = 2048
BETA = 0.25
BLK = 256
NBLK = N // BLK + E
NP = NBLK * BLK


def _sc_workers():
    try:
        info = plsc.get_sparse_core_info()
        return info.num_cores, info.num_subcores
    except Exception:
        return 2, 16


def _route_body(lab_ref, pos_ref, meta_ref):
    lab = lab_ref[...]
    r128 = lax.broadcasted_iota(jnp.int32, (128, 128), 0)
    c128 = lax.broadcasted_iota(jnp.int32, (128, 128), 1)
    tri128 = (r128 < c128).astype(jnp.float32)
    r16 = lax.broadcasted_iota(jnp.int32, (16, 16), 0)
    c16 = lax.broadcasted_iota(jnp.int32, (16, 16), 1)
    tri16 = (c16 < r16).astype(jnp.float32)

    masks, ranks, cnts = [], [], []
    for e in range(E):
        m = (lab == e).astype(jnp.float32)
        pref = lax.dot_general(m, tri128, (((1,), (0,)), ((), ())),
                               preferred_element_type=jnp.float32)
        s = jnp.sum(m, axis=1, keepdims=True)
        rowpref = lax.dot_general(tri16, s, (((1,), (0,)), ((), ())),
                                  preferred_element_type=jnp.float32)
        masks.append(m)
        ranks.append(pref + rowpref)
        cnts.append(jnp.sum(m).astype(jnp.int32))

    poffs, cumblocks = [], []
    acc = jnp.int32(0)
    for e in range(E):
        poffs.append(acc * BLK)
        acc = acc + (cnts[e] + (BLK - 1)) // BLK
        cumblocks.append(acc)

    pos = jnp.zeros((16, 128), jnp.float32)
    for e in range(E):
        pos = pos + masks[e] * (ranks[e] + poffs[e].astype(jnp.float32))
    pos_ref[...] = pos.astype(jnp.int32)

    li = lax.broadcasted_iota(jnp.int32, (E, 128), 1)
    bm = jnp.zeros((E, 128), jnp.int32)
    for e in range(E):
        bm = bm + (li >= cumblocks[e]).astype(jnp.int32)
    meta = jnp.where(li < NBLK, jnp.minimum(bm, E - 1), 0)
    for e in range(E):
        meta = meta + jnp.where(li == NBLK + e, cnts[e], 0)
        meta = meta + jnp.where(li == NBLK + E + e, poffs[e], 0)
    meta = meta + jnp.where(li == NBLK + 2 * E, cumblocks[E - 1], 0)
    meta_ref[...] = meta


def _route(lab2d):
    return pl.pallas_call(
        _route_body,
        out_shape=[
            jax.ShapeDtypeStruct((16, 128), jnp.int32),
            jax.ShapeDtypeStruct((E, 128), jnp.int32),
        ],
    )(lab2d)


def _dispatch(img, pos):
    nc, ns = _sc_workers()
    nw = nc * ns
    tpw = N // nw
    mesh = plsc.VectorSubcoreMesh(core_axis_name="c", subcore_axis_name="s")

    @functools.partial(
        pl.kernel, mesh=mesh,
        out_type=jax.ShapeDtypeStruct((NP, D), jnp.float32),
        scratch_types=[
            pltpu.VMEM((tpw,), jnp.int32),
            pltpu.VMEM((tpw, D), jnp.float32),
            pltpu.SemaphoreType.DMA,
        ],
    )
    def body(img_hbm, pos_hbm, out_hbm, idx_v, rows_v, sem):
        wid = lax.axis_index("s") * nc + lax.axis_index("c")
        base = wid * tpw
        pltpu.sync_copy(pos_hbm.at[pl.ds(base, tpw)], idx_v)
        pltpu.sync_copy(img_hbm.at[pl.ds(base, tpw)], rows_v)
        pltpu.async_copy(rows_v, out_hbm.at[idx_v], sem).wait()

    return body(img, pos)


def _gather_rows(table, idx):
    nc, ns = _sc_workers()
    nw = nc * ns
    tpw = N // nw
    mesh = plsc.VectorSubcoreMesh(core_axis_name="c", subcore_axis_name="s")

    @functools.partial(
        pl.kernel, mesh=mesh,
        out_type=jax.ShapeDtypeStruct((N, D), jnp.float32),
        scratch_types=[
            pltpu.VMEM((tpw,), jnp.int32),
            pltpu.VMEM((tpw, D), jnp.float32),
            pltpu.SemaphoreType.DMA,
        ],
    )
    def body(tab_hbm, idx_hbm, out_hbm, idx_v, rows_v, sem):
        wid = lax.axis_index("s") * nc + lax.axis_index("c")
        base = wid * tpw
        pltpu.sync_copy(idx_hbm.at[pl.ds(base, tpw)], idx_v)
        pltpu.async_copy(tab_hbm.at[idx_v], rows_v, sem).wait()
        pltpu.sync_copy(rows_v, out_hbm.at[pl.ds(base, tpw)])

    return body(table, idx)


def _main_body(m_ref, x_ref, w_ref, b_ref, cb_ref, wd_ref, bd_ref,
               dec_ref, loss_ref, sse_acc, s_acc, cbsq_ref):
    i = pl.program_id(0)
    total = m_ref[NBLK + 2 * E]

    @pl.when(i == 0)
    def _():
        sse_acc[0] = jnp.float32(0.0)
        for e in range(E):
            s_acc[e] = jnp.float32(0.0)
        cb0 = cb_ref[...]
        cbsq_ref[...] = jnp.sum(cb0 * cb0, axis=1)[None, :]

    @pl.when(i < total)
    def _():
        e = m_ref[i]
        x = x_ref[...]
        enc = lax.dot_general(x, w_ref[0], (((1,), (0,)), ((), ())),
                              preferred_element_type=jnp.float32)
        enc = enc + b_ref[pl.ds(e, 1), :]
        cb = cb_ref[...]
        xsq = jnp.sum(enc * enc, axis=1, keepdims=True)
        prod = lax.dot_general(enc, cb, (((1,), (1,)), ((), ())),
                               preferred_element_type=jnp.float32)
        d2 = xsq - 2.0 * prod + cbsq_ref[...]
        mv = jnp.min(d2, axis=1, keepdims=True)
        li = lax.broadcasted_iota(jnp.int32, (BLK, K), 1)
        idx = jnp.min(jnp.where(d2 <= mv, li, K), axis=1, keepdims=True)
        oh = (li == idx).astype(jnp.float32)
        quant = lax.dot_general(oh, cb, (((1,), (0,)), ((), ())),
                                preferred_element_type=jnp.float32)
        dec = lax.dot_general(quant, wd_ref[...], (((1,), (0,)), ((), ())),
                              preferred_element_type=jnp.float32) + bd_ref[...]
        dec_ref[...] = dec

        row = lax.broadcasted_iota(jnp.int32, (BLK, 1), 0)
        srow = i * BLK + row
        valid = (srow - m_ref[NBLK + E + e]) < m_ref[NBLK + e]
        diff = dec - x
        sse_row = jnp.sum(diff * diff, axis=1, keepdims=True)
        sse_acc[0] += jnp.sum(jnp.where(valid, sse_row, 0.0))
        qd = enc - quant
        qtok = jnp.sum(qd * qd, axis=1, keepdims=True)
        s_acc[e] += jnp.sum(jnp.where(valid, qtok, 0.0))

    @pl.when(i == NBLK - 1)
    def _():
        qloss = jnp.float32(0.0)
        for e in range(E):
            cnt = jnp.maximum(m_ref[NBLK + e].astype(jnp.float32) * D, 1.0)
            qloss += (1.0 + BETA) * s_acc[e] / cnt
        loss_ref[0] = sse_acc[0] / (N * D) + qloss / E


def _main(meta0, simg, W_enc, b_enc, codebook, W_dec, b_dec2d):
    grid_spec = pltpu.PrefetchScalarGridSpec(
        num_scalar_prefetch=1,
        grid=(NBLK,),
        in_specs=[
            pl.BlockSpec((BLK, D), lambda i, m: (i, 0)),
            pl.BlockSpec((1, D, D), lambda i, m: (m[i], 0, 0)),
            pl.BlockSpec((E, D), lambda i, m: (0, 0)),
            pl.BlockSpec((K, D), lambda i, m: (0, 0)),
            pl.BlockSpec((D, D), lambda i, m: (0, 0)),
            pl.BlockSpec((1, D), lambda i, m: (0, 0)),
        ],
        out_specs=[
            pl.BlockSpec((BLK, D), lambda i, m: (i, 0)),
            pl.BlockSpec(memory_space=pltpu.SMEM),
        ],
        scratch_shapes=[
            pltpu.SMEM((1,), jnp.float32),
            pltpu.SMEM((E,), jnp.float32),
            pltpu.VMEM((1, K), jnp.float32),
        ],
    )
    return pl.pallas_call(
        _main_body,
        grid_spec=grid_spec,
        out_shape=[
            jax.ShapeDtypeStruct((NP, D), jnp.float32),
            jax.ShapeDtypeStruct((1,), jnp.float32),
        ],
    )(meta0, simg, W_enc, b_enc, codebook, W_dec, b_dec2d)


def kernel(img, label, W_enc, b_enc, codebook, W_dec, b_dec):
    lab = label.astype(jnp.int32)
    pos2d, meta = _route(lab.reshape(16, 128))
    pos = pos2d.reshape(N)
    meta0 = meta[0]

    simg = _dispatch(img, pos)
    dec_sorted, loss = _main(meta0, simg, W_enc, b_enc, codebook,
                             W_dec, b_dec.reshape(1, D))
    out = _gather_rows(dec_sorted, pos)
    return out, loss[0]


# speedup vs baseline: 6.0887x; 1.0325x over previous
import functools

import jax
import jax.numpy as jnp
from jax import lax
from jax.experimental import pallas as pl
from jax.experimental.pallas import tpu as pltpu
from jax.experimental.pallas import tpu_sc as plsc

E = 8
D = 1024
K = 512
N = 2048
BETA = 0.25
BLK = 256
NBLK = N // BLK + E
NP = NBLK * BLK


def _sc_workers():
    try:
        info = plsc.get_sparse_core_info()
        return info.num_cores, info.num_subcores
    except Exception:
        return 2, 16


def _route_body(lab_ref, pos_ref, meta_ref):
    lab = lab_ref[...]
    r128 = lax.broadcasted_iota(jnp.int32, (128, 128), 0)
    c128 = lax.broadcasted_iota(jnp.int32, (128, 128), 1)
    tri128 = (r128 < c128).astype(jnp.float32)
    r16 = lax.broadcasted_iota(jnp.int32, (16, 16), 0)
    c16 = lax.broadcasted_iota(jnp.int32, (16, 16), 1)
    tri16 = (c16 < r16).astype(jnp.float32)

    masks, ranks, cnts = [], [], []
    for e in range(E):
        m = (lab == e).astype(jnp.float32)
        pref = lax.dot_general(m, tri128, (((1,), (0,)), ((), ())),
                               preferred_element_type=jnp.float32)
        s = jnp.sum(m, axis=1, keepdims=True)
        rowpref = lax.dot_general(tri16, s, (((1,), (0,)), ((), ())),
                                  preferred_element_type=jnp.float32)
        masks.append(m)
        ranks.append(pref + rowpref)
        cnts.append(jnp.sum(m).astype(jnp.int32))

    poffs, cumblocks = [], []
    acc = jnp.int32(0)
    for e in range(E):
        poffs.append(acc * BLK)
        acc = acc + (cnts[e] + (BLK - 1)) // BLK
        cumblocks.append(acc)

    pos = jnp.zeros((16, 128), jnp.float32)
    for e in range(E):
        pos = pos + masks[e] * (ranks[e] + poffs[e].astype(jnp.float32))
    pos_ref[...] = pos.astype(jnp.int32)

    li = lax.broadcasted_iota(jnp.int32, (E, 128), 1)
    bm = jnp.zeros((E, 128), jnp.int32)
    for e in range(E):
        bm = bm + (li >= cumblocks[e]).astype(jnp.int32)
    meta = jnp.where(li < NBLK, jnp.minimum(bm, E - 1), 0)
    for e in range(E):
        meta = meta + jnp.where(li == NBLK + e, cnts[e], 0)
        meta = meta + jnp.where(li == NBLK + E + e, poffs[e], 0)
    meta = meta + jnp.where(li == NBLK + 2 * E, cumblocks[E - 1], 0)
    meta_ref[...] = meta


def _route(lab2d):
    return pl.pallas_call(
        _route_body,
        out_shape=[
            jax.ShapeDtypeStruct((16, 128), jnp.int32),
            jax.ShapeDtypeStruct((E, 128), jnp.int32),
        ],
    )(lab2d)


def _dispatch(img, pos):
    nc, ns = _sc_workers()
    nw = nc * ns
    tpw = N // nw
    mesh = plsc.VectorSubcoreMesh(core_axis_name="c", subcore_axis_name="s")

    @functools.partial(
        pl.kernel, mesh=mesh,
        out_type=jax.ShapeDtypeStruct((NP, D), jnp.float32),
        scratch_types=[
            pltpu.VMEM((tpw,), jnp.int32),
            pltpu.VMEM((tpw, D), jnp.float32),
            pltpu.SemaphoreType.DMA,
        ],
    )
    def body(img_hbm, pos_hbm, out_hbm, idx_v, rows_v, sem):
        wid = lax.axis_index("s") * nc + lax.axis_index("c")
        base = wid * tpw
        pltpu.sync_copy(pos_hbm.at[pl.ds(base, tpw)], idx_v)
        pltpu.sync_copy(img_hbm.at[pl.ds(base, tpw)], rows_v)
        pltpu.async_copy(rows_v, out_hbm.at[idx_v], sem).wait()

    return body(img, pos)


def _gather_rows(table, idx):
    nc, ns = _sc_workers()
    nw = nc * ns
    tpw = N // nw
    mesh = plsc.VectorSubcoreMesh(core_axis_name="c", subcore_axis_name="s")

    @functools.partial(
        pl.kernel, mesh=mesh,
        out_type=jax.ShapeDtypeStruct((N, D), jnp.float32),
        scratch_types=[
            pltpu.VMEM((tpw,), jnp.int32),
            pltpu.VMEM((tpw, D), jnp.float32),
            pltpu.SemaphoreType.DMA,
        ],
    )
    def body(tab_hbm, idx_hbm, out_hbm, idx_v, rows_v, sem):
        wid = lax.axis_index("s") * nc + lax.axis_index("c")
        base = wid * tpw
        pltpu.sync_copy(idx_hbm.at[pl.ds(base, tpw)], idx_v)
        pltpu.async_copy(tab_hbm.at[idx_v], rows_v, sem).wait()
        pltpu.sync_copy(rows_v, out_hbm.at[pl.ds(base, tpw)])

    return body(table, idx)


def _main_body(m_ref, x_ref, w_ref, b_ref, cb_ref, wd_ref, bd_ref,
               dec_ref, loss_ref, sse_acc, s_acc, cbsq_ref):
    i = pl.program_id(0)
    total = m_ref[NBLK + 2 * E]

    @pl.when(i == 0)
    def _():
        sse_acc[0] = jnp.float32(0.0)
        for e in range(E):
            s_acc[e] = jnp.float32(0.0)
        cb0 = cb_ref[...]
        cbsq_ref[...] = jnp.sum(cb0 * cb0, axis=1)[None, :]

    @pl.when(i < total)
    def _():
        e = m_ref[i]
        x = x_ref[...]
        enc = lax.dot_general(x, w_ref[0], (((1,), (0,)), ((), ())),
                              preferred_element_type=jnp.float32)
        enc = enc + b_ref[pl.ds(e, 1), :]
        cb = cb_ref[...]
        xsq = jnp.sum(enc * enc, axis=1, keepdims=True)
        prod = lax.dot_general(enc, cb, (((1,), (1,)), ((), ())),
                               preferred_element_type=jnp.float32)
        d2 = xsq - 2.0 * prod + cbsq_ref[...]
        mv = jnp.min(d2, axis=1, keepdims=True)
        li = lax.broadcasted_iota(jnp.int32, (BLK, K), 1)
        idx = jnp.min(jnp.where(d2 <= mv, li, K), axis=1, keepdims=True)
        oh = (li == idx).astype(jnp.float32)
        quant = lax.dot_general(oh, cb, (((1,), (0,)), ((), ())),
                                preferred_element_type=jnp.float32)
        dec = lax.dot_general(quant, wd_ref[...], (((1,), (0,)), ((), ())),
                              preferred_element_type=jnp.float32) + bd_ref[...]
        dec_ref[...] = dec

        row = lax.broadcasted_iota(jnp.int32, (BLK, 1), 0)
        srow = i * BLK + row
        valid = (srow - m_ref[NBLK + E + e]) < m_ref[NBLK + e]
        diff = dec - x
        sse_row = jnp.sum(diff * diff, axis=1, keepdims=True)
        sse_acc[0] += jnp.sum(jnp.where(valid, sse_row, 0.0))
        qd = enc - quant
        qtok = jnp.sum(qd * qd, axis=1, keepdims=True)
        s_acc[e] += jnp.sum(jnp.where(valid, qtok, 0.0))

    @pl.when(i == NBLK - 1)
    def _():
        qloss = jnp.float32(0.0)
        for e in range(E):
            cnt = jnp.maximum(m_ref[NBLK + e].astype(jnp.float32) * D, 1.0)
            qloss += (1.0 + BETA) * s_acc[e] / cnt
        loss_ref[0] = sse_acc[0] / (N * D) + qloss / E


def _main(meta0, simg, W_enc, b_enc, codebook, W_dec, b_dec2d):
    grid_spec = pltpu.PrefetchScalarGridSpec(
        num_scalar_prefetch=1,
        grid=(NBLK,),
        in_specs=[
            pl.BlockSpec((BLK, D),
                         lambda i, m: (jnp.minimum(i, m[NBLK + 2 * E] - 1), 0)),
            pl.BlockSpec((1, D, D), lambda i, m: (m[i], 0, 0)),
            pl.BlockSpec((E, D), lambda i, m: (0, 0)),
            pl.BlockSpec((K, D), lambda i, m: (0, 0)),
            pl.BlockSpec((D, D), lambda i, m: (0, 0)),
            pl.BlockSpec((1, D), lambda i, m: (0, 0)),
        ],
        out_specs=[
            pl.BlockSpec((BLK, D),
                         lambda i, m: (jnp.minimum(i, m[NBLK + 2 * E] - 1), 0)),
            pl.BlockSpec(memory_space=pltpu.SMEM),
        ],
        scratch_shapes=[
            pltpu.SMEM((1,), jnp.float32),
            pltpu.SMEM((E,), jnp.float32),
            pltpu.VMEM((1, K), jnp.float32),
        ],
    )
    return pl.pallas_call(
        _main_body,
        grid_spec=grid_spec,
        out_shape=[
            jax.ShapeDtypeStruct((NP, D), jnp.float32),
            jax.ShapeDtypeStruct((1,), jnp.float32),
        ],
    )(meta0, simg, W_enc, b_enc, codebook, W_dec, b_dec2d)


def kernel(img, label, W_enc, b_enc, codebook, W_dec, b_dec):
    lab = label.astype(jnp.int32)
    pos2d, meta = _route(lab.reshape(16, 128))
    pos = pos2d.reshape(N)
    meta0 = meta[0]

    simg = _dispatch(img, pos)
    dec_sorted, loss = _main(meta0, simg, W_enc, b_enc, codebook,
                             W_dec, b_dec.reshape(1, D))
    out = _gather_rows(dec_sorted, pos)
    return out, loss[0]
